# Initial kernel scaffold; baseline (speedup 1.0000x reference)
import functools

import jax
import jax.numpy as jnp
from jax import lax
from jax.experimental import pallas as pl
from jax.experimental.pallas import tpu as pltpu
from jax.experimental.pallas import tpu_sc as plsc

N = 10000
E = 320000
D = 128
C = 64
B = 4096
L = 16

P_COLS = 4
CH = 4000


def kernel(nodes, edge_index, edge_weight, emb, W):
    info = plsc.get_sparse_core_info()
    NC, NS = info.num_cores, info.num_subcores
    NW = NC * NS
    EPW = E // NW
    SCOL = C // P_COLS
    M = NW // SCOL
    EPM = E // M
    NCH = EPM // CH
    BPW = B // NW

    row = edge_index[0]
    col = edge_index[1]
    mesh = plsc.VectorSubcoreMesh(core_axis_name="c", subcore_axis_name="s")

    @functools.partial(
        pl.kernel,
        out_type=jax.ShapeDtypeStruct((NW, N), jnp.float32),
        mesh=mesh,
        scratch_types=[
            pltpu.VMEM((EPW,), jnp.int32),
            pltpu.VMEM((EPW,), jnp.float32),
            pltpu.VMEM((N,), jnp.float32),
        ],
    )
    def deg_kernel(col_hbm, ew_hbm, degp_hbm, col_v, ew_v, deg_v):
        w = lax.axis_index("s") * NC + lax.axis_index("c")
        base = w * EPW
        pltpu.sync_copy(col_hbm.at[pl.ds(base, EPW)], col_v)
        pltpu.sync_copy(ew_hbm.at[pl.ds(base, EPW)], ew_v)

        def zero_body(i, _):
            deg_v[pl.ds(i * L, L)] = jnp.zeros((L,), jnp.float32)
            return _

        lax.fori_loop(0, N // L, zero_body, None)

        def edge_body(i, _):
            sl = pl.ds(i * L, L)
            plsc.addupdate_scatter(deg_v, [col_v[sl]], ew_v[sl])
            return _

        lax.fori_loop(0, EPW // L, edge_body, None)
        pltpu.sync_copy(deg_v, degp_hbm.at[w])

    degp = deg_kernel(col, edge_weight)

    wt = W.T

    def tc1_body(emb_ref, wt_ref, degp_ref, xt_ref, dinv_ref):
        xt_ref[...] = lax.dot_general(
            wt_ref[...], emb_ref[...],
            dimension_numbers=(((1,), (1,)), ((), ())),
            preferred_element_type=jnp.float32,
        )
        deg = jnp.sum(degp_ref[...], axis=0, keepdims=True) + 1.0
        dinv_ref[...] = lax.rsqrt(deg)

    xt, dinv2 = pl.pallas_call(
        tc1_body,
        out_shape=(
            jax.ShapeDtypeStruct((C, N), jnp.float32),
            jax.ShapeDtypeStruct((1, N), jnp.float32),
        ),
    )(emb, wt, degp)
    dinv = dinv2.reshape(N)

    @functools.partial(
        pl.kernel,
        out_type=jax.ShapeDtypeStruct((E,), jnp.float32),
        mesh=mesh,
        scratch_types=[
            pltpu.VMEM((N,), jnp.float32),
            pltpu.VMEM((EPW,), jnp.int32),
            pltpu.VMEM((EPW,), jnp.int32),
            pltpu.VMEM((EPW,), jnp.float32),
            pltpu.VMEM((EPW,), jnp.float32),
        ],
    )
    def norm_kernel(row_hbm, col_hbm, ew_hbm, dinv_hbm, norm_hbm,
                    dinv_v, row_v, col_v, ew_v, norm_v):
        w = lax.axis_index("s") * NC + lax.axis_index("c")
        base = w * EPW
        pltpu.sync_copy(dinv_hbm, dinv_v)
        pltpu.sync_copy(row_hbm.at[pl.ds(base, EPW)], row_v)
        pltpu.sync_copy(col_hbm.at[pl.ds(base, EPW)], col_v)
        pltpu.sync_copy(ew_hbm.at[pl.ds(base, EPW)], ew_v)

        def body(i, _):
            sl = pl.ds(i * L, L)
            dr = plsc.load_gather(dinv_v, [row_v[sl]])
            dc = plsc.load_gather(dinv_v, [col_v[sl]])
            norm_v[sl] = dr * ew_v[sl] * dc
            return _

        lax.fori_loop(0, EPW // L, body, None)
        pltpu.sync_copy(norm_v, norm_hbm.at[pl.ds(base, EPW)])

    norm = norm_kernel(row, col, edge_weight, dinv)

    @functools.partial(
        pl.kernel,
        out_type=jax.ShapeDtypeStruct((M, C, N), jnp.float32),
        mesh=mesh,
        scratch_types=[
            pltpu.VMEM((P_COLS, N), jnp.float32),
            pltpu.VMEM((P_COLS, N), jnp.float32),
            pltpu.VMEM((CH,), jnp.int32),
            pltpu.VMEM((CH,), jnp.int32),
            pltpu.VMEM((CH,), jnp.float32),
        ],
    )
    def scatter_kernel(row_hbm, col_hbm, norm_hbm, xt_hbm, outp_hbm,
                       x_v, acc_v, row_b, col_b, norm_b):
        w = lax.axis_index("s") * NC + lax.axis_index("c")
        cshard = w % SCOL
        eshard = w // SCOL
        c0 = cshard * P_COLS
        e0 = eshard * EPM
        pltpu.sync_copy(xt_hbm.at[pl.ds(c0, P_COLS), :], x_v)

        for cc in range(P_COLS):
            def zb(i, _, cc=cc):
                acc_v[cc, pl.ds(i * L, L)] = jnp.zeros((L,), jnp.float32)
                return _
            lax.fori_loop(0, N // L, zb, None)

        cidx = [jnp.full((L,), cc, jnp.int32) for cc in range(P_COLS)]

        def chunk(g, _):
            off = pl.multiple_of(e0 + g * CH, 8)
            pltpu.sync_copy(row_hbm.at[pl.ds(off, CH)], row_b)
            pltpu.sync_copy(col_hbm.at[pl.ds(off, CH)], col_b)
            pltpu.sync_copy(norm_hbm.at[pl.ds(off, CH)], norm_b)

            def inner(i, _):
                sl = pl.ds(i * L, L)
                rvec = row_b[sl]
                cvec = col_b[sl]
                nvec = norm_b[sl]
                for cc in range(P_COLS):
                    vals = plsc.load_gather(x_v, [cidx[cc], rvec])
                    plsc.addupdate_scatter(
                        acc_v, [cidx[cc], cvec], vals * nvec)
                return _

            lax.fori_loop(0, CH // L, inner, None)
            return _

        lax.fori_loop(0, NCH, chunk, None)
        pltpu.sync_copy(acc_v, outp_hbm.at[eshard, pl.ds(c0, P_COLS), :])

    outp = scatter_kernel(row, col, norm, xt)

    def tc2_body(outp_ref, xt_ref, dinv_ref, fin_ref):
        comb = xt_ref[...] * (dinv_ref[...] * dinv_ref[...])
        for m in range(M):
            comb = comb + outp_ref[m]
        fin_ref[...] = comb.T

    final = pl.pallas_call(
        tc2_body,
        out_shape=jax.ShapeDtypeStruct((N, C), jnp.float32),
    )(outp, xt, dinv2)

    @functools.partial(
        pl.kernel,
        out_type=jax.ShapeDtypeStruct((B, C), jnp.float32),
        mesh=mesh,
        scratch_types=[
            pltpu.VMEM((BPW,), jnp.int32),
            pltpu.VMEM((BPW, C), jnp.float32),
            pltpu.SemaphoreType.DMA,
        ],
    )
    def gather_kernel(fin_hbm, nodes_hbm, res_hbm, idx_v, rows_v, sem):
        w = lax.axis_index("s") * NC + lax.axis_index("c")
        base = w * BPW
        pltpu.sync_copy(nodes_hbm.at[pl.ds(base, BPW)], idx_v)
        pltpu.async_copy(fin_hbm.at[idx_v], rows_v, sem).wait()
        pltpu.sync_copy(rows_v, res_hbm.at[pl.ds(base, BPW)])

    return gather_kernel(final, nodes)


# speedup vs baseline: 15.5652x; 15.5652x over previous
import functools

import jax
import jax.numpy as jnp
from jax import lax
from jax.experimental import pallas as pl
from jax.experimental.pallas import tpu as pltpu
from jax.experimental.pallas import tpu_sc as plsc

N = 10000
E = 320000
D = 128
C = 64
B = 4096
L = 16

P_COLS = 4
CH = 4000


def kernel(nodes, edge_index, edge_weight, emb, W):
    info = plsc.get_sparse_core_info()
    NC, NS = info.num_cores, info.num_subcores
    NW = NC * NS
    EPW = E // NW
    SCOL = C // P_COLS
    M = NW // SCOL
    EPM = E // M
    NCH = EPM // CH
    BPW = B // NW

    row = edge_index[0]
    col = edge_index[1]
    mesh = plsc.VectorSubcoreMesh(core_axis_name="c", subcore_axis_name="s")

    @functools.partial(
        pl.kernel,
        out_type=jax.ShapeDtypeStruct((NW, N), jnp.float32),
        mesh=mesh,
        compiler_params=pltpu.CompilerParams(needs_layout_passes=False),
        scratch_types=[
            pltpu.VMEM((EPW,), jnp.int32),
            pltpu.VMEM((EPW,), jnp.float32),
            pltpu.VMEM((N,), jnp.float32),
        ],
    )
    def deg_kernel(col_hbm, ew_hbm, degp_hbm, col_v, ew_v, deg_v):
        w = lax.axis_index("s") * NC + lax.axis_index("c")
        base = w * EPW
        pltpu.sync_copy(col_hbm.at[pl.ds(base, EPW)], col_v)
        pltpu.sync_copy(ew_hbm.at[pl.ds(base, EPW)], ew_v)

        def zero_body(i, _):
            deg_v[pl.ds(i * L, L)] = jnp.zeros((L,), jnp.float32)
            return _

        lax.fori_loop(0, N // L, zero_body, None)

        def edge_body(i, _):
            sl = pl.ds(i * L, L)
            plsc.addupdate_scatter(deg_v, [col_v[sl]], ew_v[sl])
            return _

        lax.fori_loop(0, EPW // L, edge_body, None)
        pltpu.sync_copy(deg_v, degp_hbm.at[w])

    degp = deg_kernel(col, edge_weight)

    wt = W.T

    def tc1_body(emb_ref, wt_ref, degp_ref, xt_ref, dinv_ref):
        xt_ref[...] = lax.dot_general(
            wt_ref[...], emb_ref[...],
            dimension_numbers=(((1,), (1,)), ((), ())),
            preferred_element_type=jnp.float32,
        )
        deg = jnp.sum(degp_ref[...], axis=0, keepdims=True) + 1.0
        dinv_ref[...] = lax.rsqrt(deg)

    xt, dinv2 = pl.pallas_call(
        tc1_body,
        out_shape=(
            jax.ShapeDtypeStruct((C, N), jnp.float32),
            jax.ShapeDtypeStruct((1, N), jnp.float32),
        ),
    )(emb, wt, degp)
    dinv = dinv2.reshape(N)

    @functools.partial(
        pl.kernel,
        out_type=jax.ShapeDtypeStruct((E,), jnp.float32),
        mesh=mesh,
        compiler_params=pltpu.CompilerParams(needs_layout_passes=False),
        scratch_types=[
            pltpu.VMEM((N,), jnp.float32),
            pltpu.VMEM((EPW,), jnp.int32),
            pltpu.VMEM((EPW,), jnp.int32),
            pltpu.VMEM((EPW,), jnp.float32),
            pltpu.VMEM((EPW,), jnp.float32),
        ],
    )
    def norm_kernel(row_hbm, col_hbm, ew_hbm, dinv_hbm, norm_hbm,
                    dinv_v, row_v, col_v, ew_v, norm_v):
        w = lax.axis_index("s") * NC + lax.axis_index("c")
        base = w * EPW
        pltpu.sync_copy(dinv_hbm, dinv_v)
        pltpu.sync_copy(row_hbm.at[pl.ds(base, EPW)], row_v)
        pltpu.sync_copy(col_hbm.at[pl.ds(base, EPW)], col_v)
        pltpu.sync_copy(ew_hbm.at[pl.ds(base, EPW)], ew_v)

        def body(i, _):
            sl = pl.ds(i * L, L)
            dr = plsc.load_gather(dinv_v, [row_v[sl]])
            dc = plsc.load_gather(dinv_v, [col_v[sl]])
            norm_v[sl] = dr * ew_v[sl] * dc
            return _

        lax.fori_loop(0, EPW // L, body, None)
        pltpu.sync_copy(norm_v, norm_hbm.at[pl.ds(base, EPW)])

    norm = norm_kernel(row, col, edge_weight, dinv)

    @functools.partial(
        pl.kernel,
        out_type=jax.ShapeDtypeStruct((M, C, N), jnp.float32),
        mesh=mesh,
        compiler_params=pltpu.CompilerParams(needs_layout_passes=False),
        scratch_types=[
            pltpu.VMEM((P_COLS, N), jnp.float32),
            pltpu.VMEM((P_COLS, N), jnp.float32),
            pltpu.VMEM((CH,), jnp.int32),
            pltpu.VMEM((CH,), jnp.int32),
            pltpu.VMEM((CH,), jnp.float32),
        ],
    )
    def scatter_kernel(row_hbm, col_hbm, norm_hbm, xt_hbm, outp_hbm,
                       x_v, acc_v, row_b, col_b, norm_b):
        w = lax.axis_index("s") * NC + lax.axis_index("c")
        cshard = w % SCOL
        eshard = w // SCOL
        c0 = cshard * P_COLS
        e0 = eshard * EPM
        pltpu.sync_copy(xt_hbm.at[pl.ds(c0, P_COLS), :], x_v)

        for cc in range(P_COLS):
            def zb(i, _, cc=cc):
                acc_v[cc, pl.ds(i * L, L)] = jnp.zeros((L,), jnp.float32)
                return _
            lax.fori_loop(0, N // L, zb, None)

        cidx = [jnp.full((L,), cc, jnp.int32) for cc in range(P_COLS)]

        def chunk(g, _):
            off = pl.multiple_of(e0 + g * CH, 8)
            pltpu.sync_copy(row_hbm.at[pl.ds(off, CH)], row_b)
            pltpu.sync_copy(col_hbm.at[pl.ds(off, CH)], col_b)
            pltpu.sync_copy(norm_hbm.at[pl.ds(off, CH)], norm_b)

            def inner(i, _):
                sl = pl.ds(i * L, L)
                rvec = row_b[sl]
                cvec = col_b[sl]
                nvec = norm_b[sl]
                for cc in range(P_COLS):
                    vals = plsc.load_gather(x_v, [cidx[cc], rvec])
                    plsc.addupdate_scatter(
                        acc_v, [cidx[cc], cvec], vals * nvec)
                return _

            lax.fori_loop(0, CH // L, inner, None)
            return _

        lax.fori_loop(0, NCH, chunk, None)
        pltpu.sync_copy(acc_v, outp_hbm.at[eshard, pl.ds(c0, P_COLS), :])

    outp = scatter_kernel(row, col, norm, xt)

    def tc2_body(outp_ref, xt_ref, dinv_ref, fin_ref):
        comb = xt_ref[...] * (dinv_ref[...] * dinv_ref[...])
        for m in range(M):
            comb = comb + outp_ref[m]
        fin_ref[...] = jnp.concatenate(
            [comb.T, jnp.zeros((N, 128 - C), jnp.float32)], axis=1)

    final = pl.pallas_call(
        tc2_body,
        out_shape=jax.ShapeDtypeStruct((N, 128), jnp.float32),
    )(outp, xt, dinv2)

    @functools.partial(
        pl.kernel,
        out_type=jax.ShapeDtypeStruct((B, 128), jnp.float32),
        mesh=mesh,
        compiler_params=pltpu.CompilerParams(needs_layout_passes=False),
        scratch_types=[
            pltpu.VMEM((BPW,), jnp.int32),
            pltpu.VMEM((BPW, 128), jnp.float32),
            pltpu.SemaphoreType.DMA,
        ],
    )
    def gather_kernel(fin_hbm, nodes_hbm, res_hbm, idx_v, rows_v, sem):
        w = lax.axis_index("s") * NC + lax.axis_index("c")
        base = w * BPW
        pltpu.sync_copy(nodes_hbm.at[pl.ds(base, BPW)], idx_v)
        pltpu.async_copy(fin_hbm.at[idx_v], rows_v, sem).wait()
        pltpu.sync_copy(rows_v, res_hbm.at[pl.ds(base, BPW)])

    return gather_kernel(final, nodes)[:, :C]


# speedup vs baseline: 40.5699x; 2.6065x over previous
import functools

import jax
import jax.numpy as jnp
from jax import lax
from jax.experimental import pallas as pl
from jax.experimental.pallas import tpu as pltpu
from jax.experimental.pallas import tpu_sc as plsc

N = 10000
E = 320000
D = 128
C = 64
B = 4096
L = 16

P_COLS = 4
CH = 4000


def kernel(nodes, edge_index, edge_weight, emb, W):
    info = plsc.get_sparse_core_info()
    NC, NS = info.num_cores, info.num_subcores
    NW = NC * NS
    EPW = E // NW
    SCOL = C // P_COLS
    M = NW // SCOL
    EPM = E // M
    NCH = EPM // CH
    BPW = B // NW

    row = edge_index[0]
    col = edge_index[1]
    mesh = plsc.VectorSubcoreMesh(core_axis_name="c", subcore_axis_name="s")

    @functools.partial(
        pl.kernel,
        out_type=jax.ShapeDtypeStruct((NW, N), jnp.float32),
        mesh=mesh,
        compiler_params=pltpu.CompilerParams(needs_layout_passes=False),
        scratch_types=[
            pltpu.VMEM((EPW,), jnp.int32),
            pltpu.VMEM((EPW,), jnp.float32),
            pltpu.VMEM((N,), jnp.float32),
        ],
    )
    def deg_kernel(col_hbm, ew_hbm, degp_hbm, col_v, ew_v, deg_v):
        w = lax.axis_index("s") * NC + lax.axis_index("c")
        base = w * EPW
        pltpu.sync_copy(col_hbm.at[pl.ds(base, EPW)], col_v)
        pltpu.sync_copy(ew_hbm.at[pl.ds(base, EPW)], ew_v)

        @plsc.parallel_loop(0, N // L)
        def _zero(i):
            deg_v[pl.ds(i * L, L)] = jnp.zeros((L,), jnp.float32)

        @plsc.parallel_loop(0, EPW // L, unroll=8)
        def _edge(i):
            sl = pl.ds(i * L, L)
            plsc.addupdate_scatter(deg_v, [col_v[sl]], ew_v[sl])
        pltpu.sync_copy(deg_v, degp_hbm.at[w])

    degp = deg_kernel(col, edge_weight)

    wt = W.T

    def tc1_body(emb_ref, wt_ref, degp_ref, xt_ref, dinv_ref):
        xt_ref[...] = lax.dot_general(
            wt_ref[...], emb_ref[...],
            dimension_numbers=(((1,), (1,)), ((), ())),
            preferred_element_type=jnp.float32,
        )
        deg = jnp.sum(degp_ref[...], axis=0, keepdims=True) + 1.0
        dinv_ref[...] = lax.rsqrt(deg)

    xt, dinv2 = pl.pallas_call(
        tc1_body,
        out_shape=(
            jax.ShapeDtypeStruct((C, N), jnp.float32),
            jax.ShapeDtypeStruct((1, N), jnp.float32),
        ),
    )(emb, wt, degp)
    dinv = dinv2.reshape(N)

    @functools.partial(
        pl.kernel,
        out_type=jax.ShapeDtypeStruct((E,), jnp.float32),
        mesh=mesh,
        compiler_params=pltpu.CompilerParams(needs_layout_passes=False),
        scratch_types=[
            pltpu.VMEM((N,), jnp.float32),
            pltpu.VMEM((EPW,), jnp.int32),
            pltpu.VMEM((EPW,), jnp.int32),
            pltpu.VMEM((EPW,), jnp.float32),
            pltpu.VMEM((EPW,), jnp.float32),
        ],
    )
    def norm_kernel(row_hbm, col_hbm, ew_hbm, dinv_hbm, norm_hbm,
                    dinv_v, row_v, col_v, ew_v, norm_v):
        w = lax.axis_index("s") * NC + lax.axis_index("c")
        base = w * EPW
        pltpu.sync_copy(dinv_hbm, dinv_v)
        pltpu.sync_copy(row_hbm.at[pl.ds(base, EPW)], row_v)
        pltpu.sync_copy(col_hbm.at[pl.ds(base, EPW)], col_v)
        pltpu.sync_copy(ew_hbm.at[pl.ds(base, EPW)], ew_v)

        @plsc.parallel_loop(0, EPW // L, unroll=8)
        def _body(i):
            sl = pl.ds(i * L, L)
            dr = plsc.load_gather(dinv_v, [row_v[sl]])
            dc = plsc.load_gather(dinv_v, [col_v[sl]])
            norm_v[sl] = dr * ew_v[sl] * dc
        pltpu.sync_copy(norm_v, norm_hbm.at[pl.ds(base, EPW)])

    norm = norm_kernel(row, col, edge_weight, dinv)

    @functools.partial(
        pl.kernel,
        out_type=jax.ShapeDtypeStruct((M, C, N), jnp.float32),
        mesh=mesh,
        compiler_params=pltpu.CompilerParams(needs_layout_passes=False),
        scratch_types=[
            pltpu.VMEM((P_COLS, N), jnp.float32),
            pltpu.VMEM((P_COLS, N), jnp.float32),
            pltpu.VMEM((CH,), jnp.int32),
            pltpu.VMEM((CH,), jnp.int32),
            pltpu.VMEM((CH,), jnp.int32),
            pltpu.VMEM((CH,), jnp.int32),
            pltpu.VMEM((CH,), jnp.float32),
            pltpu.VMEM((CH,), jnp.float32),
            pltpu.SemaphoreType.DMA,
            pltpu.SemaphoreType.DMA,
        ],
    )
    def scatter_kernel(row_hbm, col_hbm, norm_hbm, xt_hbm, outp_hbm,
                       x_v, acc_v, row_b0, row_b1, col_b0, col_b1,
                       norm_b0, norm_b1, sem0, sem1):
        w = lax.axis_index("s") * NC + lax.axis_index("c")
        cshard = w % SCOL
        eshard = w // SCOL
        c0 = cshard * P_COLS
        e0 = eshard * EPM
        pltpu.sync_copy(xt_hbm.at[pl.ds(c0, P_COLS), :], x_v)

        for cc in range(P_COLS):
            @plsc.parallel_loop(0, N // L)
            def _zb(i, cc=cc):
                acc_v[cc, pl.ds(i * L, L)] = jnp.zeros((L,), jnp.float32)

        cidx = [jnp.full((L,), cc, jnp.int32) for cc in range(P_COLS)]
        bufs = ((row_b0, col_b0, norm_b0, sem0),
                (row_b1, col_b1, norm_b1, sem1))

        def start(g, slot):
            rb, cb, nb, sem = bufs[slot]
            off = pl.multiple_of(e0 + g * CH, 8)
            pltpu.async_copy(row_hbm.at[pl.ds(off, CH)], rb, sem)
            pltpu.async_copy(col_hbm.at[pl.ds(off, CH)], cb, sem)
            pltpu.async_copy(norm_hbm.at[pl.ds(off, CH)], nb, sem)

        def wait(slot):
            rb, cb, nb, sem = bufs[slot]
            pltpu.make_async_copy(row_hbm.at[pl.ds(0, CH)], rb, sem).wait()
            pltpu.make_async_copy(col_hbm.at[pl.ds(0, CH)], cb, sem).wait()
            pltpu.make_async_copy(norm_hbm.at[pl.ds(0, CH)], nb, sem).wait()

        def process(slot):
            rb, cb, nb, _ = bufs[slot]

            @plsc.parallel_loop(0, CH // L, unroll=8)
            def _inner(i):
                sl = pl.ds(i * L, L)
                rvec = rb[sl]
                cvec = cb[sl]
                nvec = nb[sl]
                for cc in range(P_COLS):
                    vals = plsc.load_gather(x_v, [cidx[cc], rvec])
                    plsc.addupdate_scatter(
                        acc_v, [cidx[cc], cvec], vals * nvec)

        start(0, 0)

        def pair(p, _):
            g0 = p * 2
            wait(0)

            @pl.when(g0 + 1 < NCH)
            def _():
                start(g0 + 1, 1)

            process(0)

            @pl.when(g0 + 1 < NCH)
            def _():
                wait(1)

                @pl.when(g0 + 2 < NCH)
                def _():
                    start(g0 + 2, 0)

                process(1)

            return _

        lax.fori_loop(0, (NCH + 1) // 2, pair, None)
        pltpu.sync_copy(acc_v, outp_hbm.at[eshard, pl.ds(c0, P_COLS), :])

    outp = scatter_kernel(row, col, norm, xt)

    def tc2_body(outp_ref, xt_ref, dinv_ref, fin_ref):
        comb = xt_ref[...] * (dinv_ref[...] * dinv_ref[...])
        for m in range(M):
            comb = comb + outp_ref[m]
        fin_ref[...] = jnp.concatenate(
            [comb.T, jnp.zeros((N, 128 - C), jnp.float32)], axis=1)

    final = pl.pallas_call(
        tc2_body,
        out_shape=jax.ShapeDtypeStruct((N, 128), jnp.float32),
    )(outp, xt, dinv2)

    @functools.partial(
        pl.kernel,
        out_type=jax.ShapeDtypeStruct((B, 128), jnp.float32),
        mesh=mesh,
        compiler_params=pltpu.CompilerParams(needs_layout_passes=False),
        scratch_types=[
            pltpu.VMEM((BPW,), jnp.int32),
            pltpu.VMEM((BPW, 128), jnp.float32),
            pltpu.SemaphoreType.DMA,
        ],
    )
    def gather_kernel(fin_hbm, nodes_hbm, res_hbm, idx_v, rows_v, sem):
        w = lax.axis_index("s") * NC + lax.axis_index("c")
        base = w * BPW
        pltpu.sync_copy(nodes_hbm.at[pl.ds(base, BPW)], idx_v)
        pltpu.async_copy(fin_hbm.at[idx_v], rows_v, sem).wait()
        pltpu.sync_copy(rows_v, res_hbm.at[pl.ds(base, BPW)])

    return gather_kernel(final, nodes)[:, :C]


# speedup vs baseline: 45.8466x; 1.1301x over previous
import functools

import jax
import jax.numpy as jnp
from jax import lax
from jax.experimental import pallas as pl
from jax.experimental.pallas import tpu as pltpu
from jax.experimental.pallas import tpu_sc as plsc

N = 10000
E = 320000
D = 128
C = 64
B = 4096
L = 16

P_COLS = 4
CH = 8000


def kernel(nodes, edge_index, edge_weight, emb, W):
    info = plsc.get_sparse_core_info()
    NC, NS = info.num_cores, info.num_subcores
    NW = NC * NS
    EPW = E // NW
    SCOL = C // P_COLS
    M = NW // SCOL
    EPM = E // M
    NCH = EPM // CH
    BPW = B // NW

    row = edge_index[0]
    col = edge_index[1]
    rc = (col << 16) | row
    mesh = plsc.VectorSubcoreMesh(core_axis_name="c", subcore_axis_name="s")

    @functools.partial(
        pl.kernel,
        out_type=jax.ShapeDtypeStruct((NW, N), jnp.float32),
        mesh=mesh,
        compiler_params=pltpu.CompilerParams(needs_layout_passes=False),
        scratch_types=[
            pltpu.VMEM((EPW,), jnp.int32),
            pltpu.VMEM((EPW,), jnp.float32),
            pltpu.VMEM((N,), jnp.float32),
        ],
    )
    def deg_kernel(rc_hbm, ew_hbm, degp_hbm, rc_v, ew_v, deg_v):
        w = lax.axis_index("s") * NC + lax.axis_index("c")
        base = w * EPW
        pltpu.sync_copy(rc_hbm.at[pl.ds(base, EPW)], rc_v)
        pltpu.sync_copy(ew_hbm.at[pl.ds(base, EPW)], ew_v)

        @plsc.parallel_loop(0, N // L)
        def _zero(i):
            deg_v[pl.ds(i * L, L)] = jnp.zeros((L,), jnp.float32)

        @plsc.parallel_loop(0, EPW // L, unroll=8)
        def _edge(i):
            sl = pl.ds(i * L, L)
            cvec = lax.shift_right_logical(rc_v[sl], 16)
            plsc.addupdate_scatter(deg_v, [cvec], ew_v[sl])
        pltpu.sync_copy(deg_v, degp_hbm.at[w])

    degp = deg_kernel(rc, edge_weight)

    wt = W.T

    def tc1_body(emb_ref, wt_ref, degp_ref, xt_ref, dinv_ref):
        xt_ref[...] = lax.dot_general(
            wt_ref[...], emb_ref[...],
            dimension_numbers=(((1,), (1,)), ((), ())),
            preferred_element_type=jnp.float32,
        )
        deg = jnp.sum(degp_ref[...], axis=0, keepdims=True) + 1.0
        dinv_ref[...] = lax.rsqrt(deg)

    xt, dinv2 = pl.pallas_call(
        tc1_body,
        out_shape=(
            jax.ShapeDtypeStruct((C, N), jnp.float32),
            jax.ShapeDtypeStruct((1, N), jnp.float32),
        ),
    )(emb, wt, degp)
    dinv = dinv2.reshape(N)
    xu = lax.bitcast_convert_type(
        xt.astype(jnp.bfloat16), jnp.uint16).astype(jnp.uint32)
    xp = lax.bitcast_convert_type(
        xu[0::2, :] | (xu[1::2, :] << 16), jnp.int32)

    @functools.partial(
        pl.kernel,
        out_type=jax.ShapeDtypeStruct((E,), jnp.float32),
        mesh=mesh,
        compiler_params=pltpu.CompilerParams(needs_layout_passes=False),
        scratch_types=[
            pltpu.VMEM((N,), jnp.float32),
            pltpu.VMEM((EPW,), jnp.int32),
            pltpu.VMEM((EPW,), jnp.float32),
            pltpu.VMEM((EPW,), jnp.float32),
        ],
    )
    def norm_kernel(rc_hbm, ew_hbm, dinv_hbm, norm_hbm,
                    dinv_v, rc_v, ew_v, norm_v):
        w = lax.axis_index("s") * NC + lax.axis_index("c")
        base = w * EPW
        pltpu.sync_copy(dinv_hbm, dinv_v)
        pltpu.sync_copy(rc_hbm.at[pl.ds(base, EPW)], rc_v)
        pltpu.sync_copy(ew_hbm.at[pl.ds(base, EPW)], ew_v)

        @plsc.parallel_loop(0, EPW // L, unroll=8)
        def _body(i):
            sl = pl.ds(i * L, L)
            rcv = rc_v[sl]
            dr = plsc.load_gather(dinv_v, [rcv & jnp.int32(0xFFFF)])
            dc = plsc.load_gather(dinv_v, [lax.shift_right_logical(rcv, 16)])
            norm_v[sl] = dr * ew_v[sl] * dc
        pltpu.sync_copy(norm_v, norm_hbm.at[pl.ds(base, EPW)])

    norm = norm_kernel(rc, edge_weight, dinv)

    @functools.partial(
        pl.kernel,
        out_type=jax.ShapeDtypeStruct((M, C, N), jnp.float32),
        mesh=mesh,
        compiler_params=pltpu.CompilerParams(needs_layout_passes=False),
        scratch_types=[
            pltpu.VMEM((P_COLS // 2, N), jnp.int32),
            pltpu.VMEM((P_COLS, N), jnp.float32),
            pltpu.VMEM((CH,), jnp.int32),
            pltpu.VMEM((CH,), jnp.int32),
            pltpu.VMEM((CH,), jnp.float32),
            pltpu.VMEM((CH,), jnp.float32),
            pltpu.SemaphoreType.DMA,
            pltpu.SemaphoreType.DMA,
        ],
    )
    def scatter_kernel(rc_hbm, norm_hbm, xp_hbm, outp_hbm,
                       x_v, acc_v, rc_b0, rc_b1,
                       norm_b0, norm_b1, sem0, sem1):
        w = lax.axis_index("s") * NC + lax.axis_index("c")
        cshard = w % SCOL
        eshard = w // SCOL
        c0 = cshard * P_COLS
        e0 = eshard * EPM
        pltpu.sync_copy(xp_hbm.at[pl.ds(cshard * (P_COLS // 2), P_COLS // 2), :],
                        x_v)

        for cc in range(P_COLS):
            @plsc.parallel_loop(0, N // L)
            def _zb(i, cc=cc):
                acc_v[cc, pl.ds(i * L, L)] = jnp.zeros((L,), jnp.float32)

        cidx = [jnp.full((L,), cc, jnp.int32) for cc in range(P_COLS)]
        pidx = [jnp.full((L,), pp, jnp.int32) for pp in range(P_COLS // 2)]
        bufs = ((rc_b0, norm_b0, sem0),
                (rc_b1, norm_b1, sem1))

        def start(g, slot):
            rb, nb, sem = bufs[slot]
            off = pl.multiple_of(e0 + g * CH, 8)
            pltpu.async_copy(rc_hbm.at[pl.ds(off, CH)], rb, sem)
            pltpu.async_copy(norm_hbm.at[pl.ds(off, CH)], nb, sem)

        def wait(slot):
            rb, nb, sem = bufs[slot]
            pltpu.make_async_copy(rc_hbm.at[pl.ds(0, CH)], rb, sem).wait()
            pltpu.make_async_copy(norm_hbm.at[pl.ds(0, CH)], nb, sem).wait()

        def process(slot):
            rb, nb, _ = bufs[slot]

            @plsc.parallel_loop(0, CH // L, unroll=8)
            def _inner(i):
                sl = pl.ds(i * L, L)
                rcv = rb[sl]
                nvec = nb[sl]
                rvec = rcv & jnp.int32(0xFFFF)
                cvec = lax.shift_right_logical(rcv, 16)
                for pp in range(P_COLS // 2):
                    pk = plsc.load_gather(x_v, [pidx[pp], rvec])
                    va = plsc.bitcast(lax.shift_left(pk, 16), jnp.float32)
                    vb = plsc.bitcast(pk & jnp.int32(-65536), jnp.float32)
                    plsc.addupdate_scatter(
                        acc_v, [cidx[2 * pp], cvec], va * nvec)
                    plsc.addupdate_scatter(
                        acc_v, [cidx[2 * pp + 1], cvec], vb * nvec)

        start(0, 0)

        def pair(p, _):
            g0 = p * 2
            wait(0)

            @pl.when(g0 + 1 < NCH)
            def _():
                start(g0 + 1, 1)

            process(0)

            @pl.when(g0 + 1 < NCH)
            def _():
                wait(1)

                @pl.when(g0 + 2 < NCH)
                def _():
                    start(g0 + 2, 0)

                process(1)

            return _

        lax.fori_loop(0, (NCH + 1) // 2, pair, None)
        pltpu.sync_copy(acc_v, outp_hbm.at[eshard, pl.ds(c0, P_COLS), :])

    outp = scatter_kernel(rc, norm, xp)

    def tc2_body(outp_ref, xt_ref, dinv_ref, fin_ref):
        comb = xt_ref[...] * (dinv_ref[...] * dinv_ref[...])
        for m in range(M):
            comb = comb + outp_ref[m]
        fin_ref[...] = jnp.concatenate(
            [comb.T, jnp.zeros((N, 128 - C), jnp.float32)], axis=1)

    final = pl.pallas_call(
        tc2_body,
        out_shape=jax.ShapeDtypeStruct((N, 128), jnp.float32),
    )(outp, xt, dinv2)

    @functools.partial(
        pl.kernel,
        out_type=jax.ShapeDtypeStruct((B, 128), jnp.float32),
        mesh=mesh,
        compiler_params=pltpu.CompilerParams(needs_layout_passes=False),
        scratch_types=[
            pltpu.VMEM((BPW,), jnp.int32),
            pltpu.VMEM((BPW, 128), jnp.float32),
            pltpu.SemaphoreType.DMA,
        ],
    )
    def gather_kernel(fin_hbm, nodes_hbm, res_hbm, idx_v, rows_v, sem):
        w = lax.axis_index("s") * NC + lax.axis_index("c")
        base = w * BPW
        pltpu.sync_copy(nodes_hbm.at[pl.ds(base, BPW)], idx_v)
        pltpu.async_copy(fin_hbm.at[idx_v], rows_v, sem).wait()
        pltpu.sync_copy(rows_v, res_hbm.at[pl.ds(base, BPW)])

    return gather_kernel(final, nodes)[:, :C]


# speedup vs baseline: 56.0038x; 1.2215x over previous
import functools

import jax
import jax.numpy as jnp
from jax import lax
from jax.experimental import pallas as pl
from jax.experimental.pallas import tpu as pltpu
from jax.experimental.pallas import tpu_sc as plsc

N = 10000
E = 320000
D = 128
C = 64
B = 4096
L = 16

P_COLS = 4


def kernel(nodes, edge_index, edge_weight, emb, W):
    info = plsc.get_sparse_core_info()
    NC, NS = info.num_cores, info.num_subcores
    NW = NC * NS
    EPW = E // NW
    SCOL = C // P_COLS
    M = NW // SCOL
    RPS = NW // M
    BPW = B // NW

    row = edge_index[0]
    col = edge_index[1]
    rc = (col << 16) | row
    mesh = plsc.VectorSubcoreMesh(core_axis_name="c", subcore_axis_name="s")

    @functools.partial(
        pl.kernel,
        out_type=(
            jax.ShapeDtypeStruct((NW, N), jnp.float32),
            jax.ShapeDtypeStruct((NW, N), jnp.float32),
        ),
        mesh=mesh,
        compiler_params=pltpu.CompilerParams(needs_layout_passes=False),
        scratch_types=[
            pltpu.VMEM((EPW,), jnp.int32),
            pltpu.VMEM((EPW,), jnp.float32),
            pltpu.VMEM((N,), jnp.float32),
            pltpu.VMEM((N,), jnp.float32),
            pltpu.VMEM((BPW,), jnp.int32),
        ],
    )
    def deg_kernel(rc_hbm, ew_hbm, nodes_hbm, degp_hbm, maskp_hbm,
                   rc_v, ew_v, deg_v, mask_v, nodes_v):
        w = lax.axis_index("s") * NC + lax.axis_index("c")
        base = w * EPW
        pltpu.sync_copy(rc_hbm.at[pl.ds(base, EPW)], rc_v)
        pltpu.sync_copy(ew_hbm.at[pl.ds(base, EPW)], ew_v)
        pltpu.sync_copy(nodes_hbm.at[pl.ds(w * BPW, BPW)], nodes_v)

        @plsc.parallel_loop(0, N // L)
        def _zero(i):
            deg_v[pl.ds(i * L, L)] = jnp.zeros((L,), jnp.float32)
            mask_v[pl.ds(i * L, L)] = jnp.zeros((L,), jnp.float32)

        ones = jnp.ones((L,), jnp.float32)

        @plsc.parallel_loop(0, BPW // L)
        def _mark(i):
            plsc.store_scatter(mask_v, [nodes_v[pl.ds(i * L, L)]], ones)

        @plsc.parallel_loop(0, EPW // L, unroll=8)
        def _edge(i):
            sl = pl.ds(i * L, L)
            cvec = lax.shift_right_logical(rc_v[sl], 16)
            plsc.addupdate_scatter(deg_v, [cvec], ew_v[sl])

        pltpu.sync_copy(deg_v, degp_hbm.at[w])
        pltpu.sync_copy(mask_v, maskp_hbm.at[w])

    degp, maskp = deg_kernel(rc, edge_weight, nodes)

    wt = W.T

    def tc1_body(emb_ref, wt_ref, degp_ref, maskp_ref,
                 xt_ref, dinv_ref, mask_ref):
        xt_ref[...] = lax.dot_general(
            wt_ref[...], emb_ref[...],
            dimension_numbers=(((1,), (1,)), ((), ())),
            preferred_element_type=jnp.float32,
        )
        deg = jnp.sum(degp_ref[...], axis=0, keepdims=True) + 1.0
        dinv_ref[...] = lax.rsqrt(deg)
        msum = jnp.sum(maskp_ref[...], axis=0, keepdims=True)
        mask_ref[...] = jnp.where(msum > 0.0, 1.0, 0.0)

    xt, dinv2, mask2 = pl.pallas_call(
        tc1_body,
        out_shape=(
            jax.ShapeDtypeStruct((C, N), jnp.float32),
            jax.ShapeDtypeStruct((1, N), jnp.float32),
            jax.ShapeDtypeStruct((1, N), jnp.float32),
        ),
    )(emb, wt, degp, maskp)
    dinv = dinv2.reshape(N)
    mask = mask2.reshape(N)
    xu = lax.bitcast_convert_type(
        xt.astype(jnp.bfloat16), jnp.uint16).astype(jnp.uint32)
    xp = lax.bitcast_convert_type(
        xu[0::2, :] | (xu[1::2, :] << 16), jnp.int32)

    @functools.partial(
        pl.kernel,
        out_type=(
            jax.ShapeDtypeStruct((E,), jnp.int32),
            jax.ShapeDtypeStruct((E,), jnp.float32),
            jax.ShapeDtypeStruct((NW, L), jnp.int32),
        ),
        mesh=mesh,
        compiler_params=pltpu.CompilerParams(needs_layout_passes=False),
        scratch_types=[
            pltpu.VMEM((N,), jnp.float32),
            pltpu.VMEM((N,), jnp.float32),
            pltpu.VMEM((EPW,), jnp.int32),
            pltpu.VMEM((EPW,), jnp.float32),
            pltpu.VMEM((EPW + L,), jnp.int32),
            pltpu.VMEM((EPW + L,), jnp.float32),
            pltpu.VMEM((L,), jnp.int32),
        ],
    )
    def filter_kernel(rc_hbm, ew_hbm, dinv_hbm, mask_hbm,
                      rcf_hbm, nmf_hbm, cnt_hbm,
                      dinv_v, mask_v, rc_v, ew_v, rcf_v, nmf_v, cnt_v):
        w = lax.axis_index("s") * NC + lax.axis_index("c")
        base = w * EPW
        pltpu.sync_copy(dinv_hbm, dinv_v)
        pltpu.sync_copy(mask_hbm, mask_v)
        pltpu.sync_copy(rc_hbm.at[pl.ds(base, EPW)], rc_v)
        pltpu.sync_copy(ew_hbm.at[pl.ds(base, EPW)], ew_v)

        def body(i, cnt):
            sl = pl.ds(i * L, L)
            rcv = rc_v[sl]
            rvec = rcv & jnp.int32(0xFFFF)
            cvec = lax.shift_right_logical(rcv, 16)
            dr = plsc.load_gather(dinv_v, [rvec])
            dc = plsc.load_gather(dinv_v, [cvec])
            nv = dr * ew_v[sl] * dc
            alive = plsc.load_gather(mask_v, [cvec]) > 0.0
            plsc.store_compressed(rcf_v.at[pl.ds(cnt, L)], rcv, mask=alive)
            plsc.store_compressed(nmf_v.at[pl.ds(cnt, L)], nv, mask=alive)
            return cnt + jnp.sum(alive.astype(jnp.int32))

        cnt = lax.fori_loop(0, EPW // L, body, jnp.int32(0))
        rcf_v[pl.ds(cnt, L)] = jnp.zeros((L,), jnp.int32)
        nmf_v[pl.ds(cnt, L)] = jnp.zeros((L,), jnp.float32)
        cnt_v[pl.ds(0, L)] = jnp.full((L,), cnt, jnp.int32)

        pltpu.sync_copy(rcf_v.at[pl.ds(0, EPW)], rcf_hbm.at[pl.ds(base, EPW)])
        pltpu.sync_copy(nmf_v.at[pl.ds(0, EPW)], nmf_hbm.at[pl.ds(base, EPW)])
        pltpu.sync_copy(cnt_v, cnt_hbm.at[w])

    rcf, nmf, counts = filter_kernel(rc, edge_weight, dinv, mask)

    @functools.partial(
        pl.kernel,
        out_type=jax.ShapeDtypeStruct((M, C, N), jnp.float32),
        mesh=mesh,
        compiler_params=pltpu.CompilerParams(needs_layout_passes=False),
        scratch_types=[
            pltpu.VMEM((P_COLS // 2, N), jnp.int32),
            pltpu.VMEM((P_COLS, N), jnp.float32),
            pltpu.VMEM((NW, L), jnp.int32),
            pltpu.VMEM((EPW,), jnp.int32),
            pltpu.VMEM((EPW,), jnp.int32),
            pltpu.VMEM((EPW,), jnp.float32),
            pltpu.VMEM((EPW,), jnp.float32),
            pltpu.SemaphoreType.DMA,
            pltpu.SemaphoreType.DMA,
        ],
    )
    def scatter_kernel(rcf_hbm, nmf_hbm, cnt_hbm, xp_hbm, outp_hbm,
                       x_v, acc_v, cnt_v, rc_b0, rc_b1,
                       nm_b0, nm_b1, sem0, sem1):
        w = lax.axis_index("s") * NC + lax.axis_index("c")
        cshard = w % SCOL
        eshard = w // SCOL
        c0 = cshard * P_COLS
        pltpu.sync_copy(
            xp_hbm.at[pl.ds(cshard * (P_COLS // 2), P_COLS // 2), :], x_v)
        pltpu.sync_copy(cnt_hbm, cnt_v)

        for cc in range(P_COLS):
            @plsc.parallel_loop(0, N // L)
            def _zb(i, cc=cc):
                acc_v[cc, pl.ds(i * L, L)] = jnp.zeros((L,), jnp.float32)

        cidx = [jnp.full((L,), cc, jnp.int32) for cc in range(P_COLS)]
        pidx = [jnp.full((L,), pp, jnp.int32) for pp in range(P_COLS // 2)]
        bufs = ((rc_b0, nm_b0, sem0), (rc_b1, nm_b1, sem1))

        def start(r, slot):
            rb, nb, sem = bufs[slot]
            off = (eshard * RPS + r) * EPW
            pltpu.async_copy(rcf_hbm.at[pl.ds(off, EPW)], rb, sem)
            pltpu.async_copy(nmf_hbm.at[pl.ds(off, EPW)], nb, sem)

        def wait(slot):
            rb, nb, sem = bufs[slot]
            pltpu.make_async_copy(rcf_hbm.at[pl.ds(0, EPW)], rb, sem).wait()
            pltpu.make_async_copy(nmf_hbm.at[pl.ds(0, EPW)], nb, sem).wait()

        start(0, 0)
        for r in range(RPS):
            slot = r % 2
            wait(slot)
            if r + 1 < RPS:
                start(r + 1, 1 - slot)
            rb, nb, _ = bufs[slot]
            cnt = cnt_v[eshard * RPS + r, pl.ds(0, L)][0]
            ngrp = (cnt + (L - 1)) // L

            @plsc.parallel_loop(0, ngrp, unroll=8)
            def _inner(i, rb=rb, nb=nb):
                sl = pl.ds(i * L, L)
                rcv = rb[sl]
                nvec = nb[sl]
                rvec = rcv & jnp.int32(0xFFFF)
                cvec = lax.shift_right_logical(rcv, 16)
                for pp in range(P_COLS // 2):
                    pk = plsc.load_gather(x_v, [pidx[pp], rvec])
                    va = plsc.bitcast(lax.shift_left(pk, 16), jnp.float32)
                    vb = plsc.bitcast(pk & jnp.int32(-65536), jnp.float32)
                    plsc.addupdate_scatter(
                        acc_v, [cidx[2 * pp], cvec], va * nvec)
                    plsc.addupdate_scatter(
                        acc_v, [cidx[2 * pp + 1], cvec], vb * nvec)

        pltpu.sync_copy(acc_v, outp_hbm.at[eshard, pl.ds(c0, P_COLS), :])

    outp = scatter_kernel(rcf, nmf, counts, xp)

    def tc2_body(outp_ref, xt_ref, dinv_ref, fin_ref):
        comb = xt_ref[...] * (dinv_ref[...] * dinv_ref[...])
        for m in range(M):
            comb = comb + outp_ref[m]
        fin_ref[...] = jnp.concatenate(
            [comb.T, jnp.zeros((N, 128 - C), jnp.float32)], axis=1)

    final = pl.pallas_call(
        tc2_body,
        out_shape=jax.ShapeDtypeStruct((N, 128), jnp.float32),
    )(outp, xt, dinv2)

    @functools.partial(
        pl.kernel,
        out_type=jax.ShapeDtypeStruct((B, 128), jnp.float32),
        mesh=mesh,
        compiler_params=pltpu.CompilerParams(needs_layout_passes=False),
        scratch_types=[
            pltpu.VMEM((BPW,), jnp.int32),
            pltpu.VMEM((BPW, 128), jnp.float32),
            pltpu.SemaphoreType.DMA,
        ],
    )
    def gather_kernel(fin_hbm, nodes_hbm, res_hbm, idx_v, rows_v, sem):
        w = lax.axis_index("s") * NC + lax.axis_index("c")
        base = w * BPW
        pltpu.sync_copy(nodes_hbm.at[pl.ds(base, BPW)], idx_v)
        pltpu.async_copy(fin_hbm.at[idx_v], rows_v, sem).wait()
        pltpu.sync_copy(rows_v, res_hbm.at[pl.ds(base, BPW)])

    return gather_kernel(final, nodes)[:, :C]


# speedup vs baseline: 57.5069x; 1.0268x over previous
import functools

import jax
import jax.numpy as jnp
from jax import lax
from jax.experimental import pallas as pl
from jax.experimental.pallas import tpu as pltpu
from jax.experimental.pallas import tpu_sc as plsc

N = 10000
E = 320000
D = 128
C = 64
B = 4096
L = 16

P_COLS = 4


def kernel(nodes, edge_index, edge_weight, emb, W):
    info = plsc.get_sparse_core_info()
    NC, NS = info.num_cores, info.num_subcores
    NW = NC * NS
    EPW = E // NW
    SCOL = C // P_COLS
    M = NW // SCOL
    RPS = NW // M
    BPW = B // NW

    row = edge_index[0]
    col = edge_index[1]
    rc = (col << 16) | row
    mesh = plsc.VectorSubcoreMesh(core_axis_name="c", subcore_axis_name="s")

    @functools.partial(
        pl.kernel,
        out_type=(
            jax.ShapeDtypeStruct((NW, N), jnp.float32),
            jax.ShapeDtypeStruct((NW, N), jnp.float32),
        ),
        mesh=mesh,
        compiler_params=pltpu.CompilerParams(needs_layout_passes=False),
        scratch_types=[
            pltpu.VMEM((EPW,), jnp.int32),
            pltpu.VMEM((EPW,), jnp.float32),
            pltpu.VMEM((N,), jnp.float32),
            pltpu.VMEM((N,), jnp.float32),
            pltpu.VMEM((BPW,), jnp.int32),
        ],
    )
    def deg_kernel(rc_hbm, ew_hbm, nodes_hbm, degp_hbm, maskp_hbm,
                   rc_v, ew_v, deg_v, mask_v, nodes_v):
        w = lax.axis_index("s") * NC + lax.axis_index("c")
        base = w * EPW
        pltpu.sync_copy(rc_hbm.at[pl.ds(base, EPW)], rc_v)
        pltpu.sync_copy(ew_hbm.at[pl.ds(base, EPW)], ew_v)
        pltpu.sync_copy(nodes_hbm.at[pl.ds(w * BPW, BPW)], nodes_v)

        @plsc.parallel_loop(0, N // L)
        def _zero(i):
            deg_v[pl.ds(i * L, L)] = jnp.zeros((L,), jnp.float32)
            mask_v[pl.ds(i * L, L)] = jnp.zeros((L,), jnp.float32)

        ones = jnp.ones((L,), jnp.float32)

        @plsc.parallel_loop(0, BPW // L)
        def _mark(i):
            plsc.store_scatter(mask_v, [nodes_v[pl.ds(i * L, L)]], ones)

        @plsc.parallel_loop(0, EPW // L, unroll=8)
        def _edge(i):
            sl = pl.ds(i * L, L)
            cvec = lax.shift_right_logical(rc_v[sl], 16)
            plsc.addupdate_scatter(deg_v, [cvec], ew_v[sl])

        pltpu.sync_copy(deg_v, degp_hbm.at[w])
        pltpu.sync_copy(mask_v, maskp_hbm.at[w])

    degp, maskp = deg_kernel(rc, edge_weight, nodes)

    wt = W.T
    wt_even = wt[0::2]
    wt_odd = wt[1::2]

    def tcmm_body(emb_ref, wt_ref, wte_ref, wto_ref, xt_ref, xp_ref):
        dn = (((1,), (1,)), ((), ()))
        xt_ref[...] = lax.dot_general(
            wt_ref[...], emb_ref[...], dimension_numbers=dn,
            preferred_element_type=jnp.float32)
        lo = lax.dot_general(
            wte_ref[...], emb_ref[...], dimension_numbers=dn,
            preferred_element_type=jnp.float32)
        hi = lax.dot_general(
            wto_ref[...], emb_ref[...], dimension_numbers=dn,
            preferred_element_type=jnp.float32)
        lou = lax.bitcast_convert_type(
            lo.astype(jnp.bfloat16), jnp.uint16).astype(jnp.uint32)
        hiu = lax.bitcast_convert_type(
            hi.astype(jnp.bfloat16), jnp.uint16).astype(jnp.uint32)
        xp_ref[...] = lax.bitcast_convert_type(lou | (hiu << 16), jnp.int32)

    xt, xp = pl.pallas_call(
        tcmm_body,
        out_shape=(
            jax.ShapeDtypeStruct((C, N), jnp.float32),
            jax.ShapeDtypeStruct((C // 2, N), jnp.int32),
        ),
    )(emb, wt, wt_even, wt_odd)

    def tcsm_body(degp_ref, maskp_ref, dinv_ref, mask_ref):
        deg = jnp.sum(degp_ref[...], axis=0, keepdims=True) + 1.0
        dinv_ref[...] = lax.rsqrt(deg)
        msum = jnp.sum(maskp_ref[...], axis=0, keepdims=True)
        mask_ref[...] = jnp.where(msum > 0.0, 1.0, 0.0)

    dinv2, mask2 = pl.pallas_call(
        tcsm_body,
        out_shape=(
            jax.ShapeDtypeStruct((1, N), jnp.float32),
            jax.ShapeDtypeStruct((1, N), jnp.float32),
        ),
    )(degp, maskp)
    dinv = dinv2.reshape(N)
    mask = mask2.reshape(N)

    @functools.partial(
        pl.kernel,
        out_type=(
            jax.ShapeDtypeStruct((E,), jnp.int32),
            jax.ShapeDtypeStruct((E,), jnp.float32),
            jax.ShapeDtypeStruct((NW, L), jnp.int32),
        ),
        mesh=mesh,
        compiler_params=pltpu.CompilerParams(needs_layout_passes=False),
        scratch_types=[
            pltpu.VMEM((N,), jnp.float32),
            pltpu.VMEM((N,), jnp.float32),
            pltpu.VMEM((EPW,), jnp.int32),
            pltpu.VMEM((EPW,), jnp.float32),
            pltpu.VMEM((EPW + L,), jnp.int32),
            pltpu.VMEM((EPW + L,), jnp.float32),
            pltpu.VMEM((L,), jnp.int32),
        ],
    )
    def filter_kernel(rc_hbm, ew_hbm, dinv_hbm, mask_hbm,
                      rcf_hbm, nmf_hbm, cnt_hbm,
                      dinv_v, mask_v, rc_v, ew_v, rcf_v, nmf_v, cnt_v):
        w = lax.axis_index("s") * NC + lax.axis_index("c")
        base = w * EPW
        pltpu.sync_copy(dinv_hbm, dinv_v)
        pltpu.sync_copy(mask_hbm, mask_v)
        pltpu.sync_copy(rc_hbm.at[pl.ds(base, EPW)], rc_v)
        pltpu.sync_copy(ew_hbm.at[pl.ds(base, EPW)], ew_v)

        def body(i, cnt):
            sl = pl.ds(i * L, L)
            rcv = rc_v[sl]
            rvec = rcv & jnp.int32(0xFFFF)
            cvec = lax.shift_right_logical(rcv, 16)
            dr = plsc.load_gather(dinv_v, [rvec])
            dc = plsc.load_gather(dinv_v, [cvec])
            nv = dr * ew_v[sl] * dc
            alive = plsc.load_gather(mask_v, [cvec]) > 0.0
            plsc.store_compressed(rcf_v.at[pl.ds(cnt, L)], rcv, mask=alive)
            plsc.store_compressed(nmf_v.at[pl.ds(cnt, L)], nv, mask=alive)
            return cnt + jnp.sum(alive.astype(jnp.int32))

        cnt = lax.fori_loop(0, EPW // L, body, jnp.int32(0))
        rcf_v[pl.ds(cnt, L)] = jnp.zeros((L,), jnp.int32)
        nmf_v[pl.ds(cnt, L)] = jnp.zeros((L,), jnp.float32)
        cnt_v[pl.ds(0, L)] = jnp.full((L,), cnt, jnp.int32)

        pltpu.sync_copy(rcf_v.at[pl.ds(0, EPW)], rcf_hbm.at[pl.ds(base, EPW)])
        pltpu.sync_copy(nmf_v.at[pl.ds(0, EPW)], nmf_hbm.at[pl.ds(base, EPW)])
        pltpu.sync_copy(cnt_v, cnt_hbm.at[w])

    rcf, nmf, counts = filter_kernel(rc, edge_weight, dinv, mask)

    @functools.partial(
        pl.kernel,
        out_type=jax.ShapeDtypeStruct((M, C, N), jnp.float32),
        mesh=mesh,
        compiler_params=pltpu.CompilerParams(needs_layout_passes=False),
        scratch_types=[
            pltpu.VMEM((P_COLS // 2, N), jnp.int32),
            pltpu.VMEM((P_COLS, N), jnp.float32),
            pltpu.VMEM((NW, L), jnp.int32),
            pltpu.VMEM((EPW,), jnp.int32),
            pltpu.VMEM((EPW,), jnp.int32),
            pltpu.VMEM((EPW,), jnp.float32),
            pltpu.VMEM((EPW,), jnp.float32),
            pltpu.SemaphoreType.DMA,
            pltpu.SemaphoreType.DMA,
        ],
    )
    def scatter_kernel(rcf_hbm, nmf_hbm, cnt_hbm, xp_hbm, outp_hbm,
                       x_v, acc_v, cnt_v, rc_b0, rc_b1,
                       nm_b0, nm_b1, sem0, sem1):
        w = lax.axis_index("s") * NC + lax.axis_index("c")
        cshard = w % SCOL
        eshard = w // SCOL
        c0 = cshard * P_COLS
        pltpu.sync_copy(
            xp_hbm.at[pl.ds(cshard * (P_COLS // 2), P_COLS // 2), :], x_v)
        pltpu.sync_copy(cnt_hbm, cnt_v)

        for cc in range(P_COLS):
            @plsc.parallel_loop(0, N // L)
            def _zb(i, cc=cc):
                acc_v[cc, pl.ds(i * L, L)] = jnp.zeros((L,), jnp.float32)

        cidx = [jnp.full((L,), cc, jnp.int32) for cc in range(P_COLS)]
        pidx = [jnp.full((L,), pp, jnp.int32) for pp in range(P_COLS // 2)]
        bufs = ((rc_b0, nm_b0, sem0), (rc_b1, nm_b1, sem1))

        def start(r, slot):
            rb, nb, sem = bufs[slot]
            off = (eshard * RPS + r) * EPW
            pltpu.async_copy(rcf_hbm.at[pl.ds(off, EPW)], rb, sem)
            pltpu.async_copy(nmf_hbm.at[pl.ds(off, EPW)], nb, sem)

        def wait(slot):
            rb, nb, sem = bufs[slot]
            pltpu.make_async_copy(rcf_hbm.at[pl.ds(0, EPW)], rb, sem).wait()
            pltpu.make_async_copy(nmf_hbm.at[pl.ds(0, EPW)], nb, sem).wait()

        start(0, 0)
        for r in range(RPS):
            slot = r % 2
            wait(slot)
            if r + 1 < RPS:
                start(r + 1, 1 - slot)
            rb, nb, _ = bufs[slot]
            cnt = cnt_v[eshard * RPS + r, pl.ds(0, L)][0]
            ngrp = (cnt + (L - 1)) // L

            @plsc.parallel_loop(0, ngrp, unroll=8)
            def _inner(i, rb=rb, nb=nb):
                sl = pl.ds(i * L, L)
                rcv = rb[sl]
                nvec = nb[sl]
                rvec = rcv & jnp.int32(0xFFFF)
                cvec = lax.shift_right_logical(rcv, 16)
                for pp in range(P_COLS // 2):
                    pk = plsc.load_gather(x_v, [pidx[pp], rvec])
                    va = plsc.bitcast(lax.shift_left(pk, 16), jnp.float32)
                    vb = plsc.bitcast(pk & jnp.int32(-65536), jnp.float32)
                    plsc.addupdate_scatter(
                        acc_v, [cidx[2 * pp], cvec], va * nvec)
                    plsc.addupdate_scatter(
                        acc_v, [cidx[2 * pp + 1], cvec], vb * nvec)

        pltpu.sync_copy(acc_v, outp_hbm.at[eshard, pl.ds(c0, P_COLS), :])

    outp = scatter_kernel(rcf, nmf, counts, xp)

    def tc2_body(outp_ref, xt_ref, dinv_ref, fin_ref):
        comb = xt_ref[...] * (dinv_ref[...] * dinv_ref[...])
        for m in range(M):
            comb = comb + outp_ref[m]
        fin_ref[...] = jnp.concatenate(
            [comb.T, jnp.zeros((N, 128 - C), jnp.float32)], axis=1)

    final = pl.pallas_call(
        tc2_body,
        out_shape=jax.ShapeDtypeStruct((N, 128), jnp.float32),
    )(outp, xt, dinv2)

    @functools.partial(
        pl.kernel,
        out_type=jax.ShapeDtypeStruct((B, 128), jnp.float32),
        mesh=mesh,
        compiler_params=pltpu.CompilerParams(needs_layout_passes=False),
        scratch_types=[
            pltpu.VMEM((BPW,), jnp.int32),
            pltpu.VMEM((BPW, 128), jnp.float32),
            pltpu.SemaphoreType.DMA,
        ],
    )
    def gather_kernel(fin_hbm, nodes_hbm, res_hbm, idx_v, rows_v, sem):
        w = lax.axis_index("s") * NC + lax.axis_index("c")
        base = w * BPW
        pltpu.sync_copy(nodes_hbm.at[pl.ds(base, BPW)], idx_v)
        pltpu.async_copy(fin_hbm.at[idx_v], rows_v, sem).wait()
        pltpu.sync_copy(rows_v, res_hbm.at[pl.ds(base, BPW)])

    return gather_kernel(final, nodes)[:, :C]
